# baseline (speedup 1.0000x reference)
import functools

import jax
import jax.numpy as jnp
from jax import lax
from jax.experimental import pallas as pl
from jax.experimental.pallas import tpu as pltpu
from jax.experimental.pallas import tpu_sc as plsc


def _enc_vq_kernel(x_ref, w1_ref, b1_ref, w2_ref, b2_ref, w3_ref, b3_ref,
                   embt_ref, e2h_ref, ze_ref, idx_ref):
    h = jnp.dot(x_ref[...], w1_ref[...], preferred_element_type=jnp.float32)
    h = jnp.maximum(h + b1_ref[...], 0.0)
    h = jnp.dot(h.astype(jnp.bfloat16), w2_ref[...],
                preferred_element_type=jnp.float32)
    h = jnp.maximum(h + b2_ref[...], 0.0)
    z = jnp.dot(h.astype(jnp.bfloat16), w3_ref[...],
                preferred_element_type=jnp.float32)
    z = z + b3_ref[...]
    ze_ref[...] = z
    mm = jnp.dot(z.astype(jnp.bfloat16), embt_ref[...],
                 preferred_element_type=jnp.float32)
    s = e2h_ref[...] - mm
    idx = jnp.argmin(s, axis=1).astype(jnp.int32)
    idx_ref[...] = idx.reshape(idx_ref.shape)


def _dec_kernel(zq_ref, ze_ref, w1_ref, b1_ref, w2_ref, b2_ref, w3_ref, b3_ref,
                out_ref, ls_ref):
    zq = zq_ref[...]
    d = ze_ref[...] - zq.astype(jnp.float32)
    ls_ref[...] = jnp.sum(d * d, axis=0, keepdims=True).reshape(ls_ref.shape)
    h = jnp.dot(zq, w1_ref[...],
                preferred_element_type=jnp.float32)
    h = jnp.maximum(h + b1_ref[...], 0.0)
    h = jnp.dot(h.astype(jnp.bfloat16), w2_ref[...],
                preferred_element_type=jnp.float32)
    h = jnp.maximum(h + b2_ref[...], 0.0)
    o = jnp.dot(h.astype(jnp.bfloat16), w3_ref[...],
                preferred_element_type=jnp.float32)
    out_ref[...] = o + b3_ref[...]


def _gather_rows(table, idx):
    V, ln = table.shape
    (n,) = idx.shape
    info = pltpu.get_tpu_info().sparse_core
    nw = info.num_cores * info.num_subcores
    b_per_w = n // nw
    mesh = plsc.VectorSubcoreMesh(core_axis_name="c", subcore_axis_name="s")

    @functools.partial(
        pl.kernel, mesh=mesh,
        out_type=jax.ShapeDtypeStruct((n, ln), table.dtype),
        scratch_types=[
            pltpu.VMEM((b_per_w,), jnp.int32),
            pltpu.VMEM((b_per_w, ln), table.dtype),
            pltpu.SemaphoreType.DMA,
        ],
    )
    def k(table_hbm, idx_hbm, out_hbm, idx_v, rows_v, sem):
        wid = lax.axis_index("s") * info.num_cores + lax.axis_index("c")
        base = wid * b_per_w
        pltpu.sync_copy(idx_hbm.at[pl.ds(base, b_per_w)], idx_v)
        k = 8
        ch = b_per_w // k
        copies = []
        for b in range(k):
            copies.append(pltpu.make_async_copy(
                table_hbm.at[idx_v.at[pl.ds(b * ch, ch)]],
                rows_v.at[pl.ds(b * ch, ch)], sem))
        for c in copies:
            c.start()
        for c in copies:
            c.wait()
        pltpu.sync_copy(rows_v, out_hbm.at[pl.ds(base, b_per_w)])

    return k(table, idx)


def kernel(x, embedding, ew1, eb1, ew2, eb2, ew3, eb3,
           dw1, db1, dw2, db2, dw3, db3):
    Bb, Tt, A = x.shape
    N = Bb * Tt
    K, D = embedding.shape
    H = ew1.shape[0]
    f32 = jnp.float32
    bf16 = jnp.bfloat16

    xf = x.reshape(N, A).astype(bf16)
    ew1t = ew1.T.astype(bf16)
    ew2t = ew2.T.astype(bf16)
    ew3t = ew3.T.astype(bf16)
    dw1t = dw1.T.astype(bf16)
    dw2t = dw2.T.astype(bf16)
    dw3t = dw3.T.astype(bf16)
    embt = embedding.T.astype(bf16)
    emb_packed = lax.bitcast_convert_type(
        embedding.astype(bf16).reshape(K, D // 2, 2), jnp.int32)
    e2h = (0.5 * jnp.sum(embedding * embedding, axis=1)).reshape(1, K)

    ze, idx = _encode_stage(xf, ew1t, eb1, ew2t, eb2, ew3t, eb3, embt, e2h,
                            N, A, H, D, K)

    zq_packed = _gather_rows(emb_packed, idx.reshape(N))
    zq = lax.bitcast_convert_type(zq_packed, bf16).reshape(N, D)

    BM2 = 512
    grid2 = N // BM2
    full = lambda shape: pl.BlockSpec(shape, lambda i: (0, 0))
    out, lparts = pl.pallas_call(
        _dec_kernel,
        grid=(grid2,),
        in_specs=[
            pl.BlockSpec((BM2, D), lambda i: (i, 0)),
            pl.BlockSpec((BM2, D), lambda i: (i, 0)),
            full((D, H)),
            full((1, H)),
            full((H, H)),
            full((1, H)),
            full((H, A)),
            full((1, A)),
        ],
        out_specs=[
            pl.BlockSpec((BM2, A), lambda i: (i, 0)),
            pl.BlockSpec((1, 1, D), lambda i: (i, 0, 0)),
        ],
        out_shape=[
            jax.ShapeDtypeStruct((N, A), jnp.float32),
            jax.ShapeDtypeStruct((grid2, 1, D), jnp.float32),
        ],
        compiler_params=pltpu.CompilerParams(
            dimension_semantics=("parallel",)),
    )(zq, ze, dw1t, db1.reshape(1, H), dw2t, db2.reshape(1, H),
      dw3t, db3.reshape(1, A))

    loss = jnp.sum(lparts) / (N * D)
    return (out.reshape(Bb, Tt, A), loss, loss,
            idx.reshape(Bb, Tt).astype(jnp.int32))


def _encode_stage(xf, ew1t, eb1, ew2t, eb2, ew3t, eb3, embt, e2h,
                  N, A, H, D, K):
    BM = 256
    grid = N // BM
    full = lambda shape: pl.BlockSpec(shape, lambda i: (0, 0))
    return pl.pallas_call(
        _enc_vq_kernel,
        grid=(grid,),
        in_specs=[
            pl.BlockSpec((BM, A), lambda i: (i, 0)),
            full((A, H)),
            full((1, H)),
            full((H, H)),
            full((1, H)),
            full((H, D)),
            full((1, D)),
            full((D, K)),
            full((1, K)),
        ],
        out_specs=[
            pl.BlockSpec((BM, D), lambda i: (i, 0)),
            pl.BlockSpec((BM, 1), lambda i: (i, 0)),
        ],
        out_shape=[
            jax.ShapeDtypeStruct((N, D), jnp.float32),
            jax.ShapeDtypeStruct((N, 1), jnp.int32),
        ],
        compiler_params=pltpu.CompilerParams(
            dimension_semantics=("parallel",)),
    )(xf, ew1t, eb1.reshape(1, H), ew2t, eb2.reshape(1, H),
      ew3t, eb3.reshape(1, D), embt, e2h)


# speedup vs baseline: 1.7100x; 1.7100x over previous
import functools

import jax
import jax.numpy as jnp
from jax import lax
from jax.experimental import pallas as pl
from jax.experimental.pallas import tpu as pltpu
from jax.experimental.pallas import tpu_sc as plsc

SC_ROWS = 2048


def _enc_vq_kernel(x_ref, w1_ref, b1_ref, w2_ref, b2_ref, w3_ref, b3_ref,
                   embt_ref, e2h_ref, ze_ref, idx_ref):
    h = jnp.dot(x_ref[...], w1_ref[...], preferred_element_type=jnp.float32)
    h = jnp.maximum(h + b1_ref[...], 0.0)
    h = jnp.dot(h.astype(jnp.bfloat16), w2_ref[...],
                preferred_element_type=jnp.float32)
    h = jnp.maximum(h + b2_ref[...], 0.0)
    z = jnp.dot(h.astype(jnp.bfloat16), w3_ref[...],
                preferred_element_type=jnp.float32)
    z = z + b3_ref[...]
    ze_ref[...] = z
    mm = jnp.dot(z.astype(jnp.bfloat16), embt_ref[...],
                 preferred_element_type=jnp.float32)
    s = e2h_ref[...] - mm
    idx = jnp.argmin(s, axis=1).astype(jnp.int32)
    idx_ref[...] = idx.reshape(idx_ref.shape)


def _dec_body(zq, ze_ref, w1_ref, b1_ref, w2_ref, b2_ref, w3_ref, b3_ref,
              out_ref, ls_ref):
    d = ze_ref[...] - zq.astype(jnp.float32)
    ls_ref[...] = jnp.sum(d * d, axis=0, keepdims=True).reshape(ls_ref.shape)
    h = jnp.dot(zq, w1_ref[...], preferred_element_type=jnp.float32)
    h = jnp.maximum(h + b1_ref[...], 0.0)
    h = jnp.dot(h.astype(jnp.bfloat16), w2_ref[...],
                preferred_element_type=jnp.float32)
    h = jnp.maximum(h + b2_ref[...], 0.0)
    o = jnp.dot(h.astype(jnp.bfloat16), w3_ref[...],
                preferred_element_type=jnp.float32)
    out_ref[...] = o + b3_ref[...]


def _dec_zq_kernel(zq_ref, ze_ref, w1_ref, b1_ref, w2_ref, b2_ref,
                   w3_ref, b3_ref, out_ref, ls_ref):
    _dec_body(zq_ref[...], ze_ref, w1_ref, b1_ref, w2_ref, b2_ref,
              w3_ref, b3_ref, out_ref, ls_ref)


def _dec_onehot_kernel(idx_ref, ze_ref, embn_ref, w1_ref, b1_ref, w2_ref,
                       b2_ref, w3_ref, b3_ref, out_ref, ls_ref):
    bm, _ = idx_ref.shape
    kk = embn_ref.shape[0]
    iota = lax.broadcasted_iota(jnp.int32, (bm, kk), 1)
    oh = (iota == idx_ref[...]).astype(jnp.bfloat16)
    zq = jnp.dot(oh, embn_ref[...], preferred_element_type=jnp.float32)
    _dec_body(zq.astype(jnp.bfloat16), ze_ref, w1_ref, b1_ref, w2_ref,
              b2_ref, w3_ref, b3_ref, out_ref, ls_ref)


def _gather_rows(table, idx):
    V, ln = table.shape
    (n,) = idx.shape
    info = pltpu.get_tpu_info().sparse_core
    nw = info.num_cores * info.num_subcores
    b_per_w = n // nw
    mesh = plsc.VectorSubcoreMesh(core_axis_name="c", subcore_axis_name="s")

    @functools.partial(
        pl.kernel, mesh=mesh,
        out_type=jax.ShapeDtypeStruct((n, ln), table.dtype),
        scratch_types=[
            pltpu.VMEM((b_per_w,), jnp.int32),
            pltpu.VMEM((b_per_w, ln), table.dtype),
            pltpu.SemaphoreType.DMA,
        ],
    )
    def k(table_hbm, idx_hbm, out_hbm, idx_v, rows_v, sem):
        wid = lax.axis_index("s") * info.num_cores + lax.axis_index("c")
        base = wid * b_per_w
        pltpu.sync_copy(idx_hbm.at[pl.ds(base, b_per_w)], idx_v)
        pltpu.async_copy(table_hbm.at[idx_v], rows_v, sem).wait()
        pltpu.sync_copy(rows_v, out_hbm.at[pl.ds(base, b_per_w)])

    return k(table, idx)


def kernel(x, embedding, ew1, eb1, ew2, eb2, ew3, eb3,
           dw1, db1, dw2, db2, dw3, db3):
    Bb, Tt, A = x.shape
    N = Bb * Tt
    K, D = embedding.shape
    H = ew1.shape[0]
    bf16 = jnp.bfloat16

    xf = x.reshape(N, A).astype(bf16)
    ew1t = ew1.T.astype(bf16)
    ew2t = ew2.T.astype(bf16)
    ew3t = ew3.T.astype(bf16)
    dw1t = dw1.T.astype(bf16)
    dw2t = dw2.T.astype(bf16)
    dw3t = dw3.T.astype(bf16)
    embt = embedding.T.astype(bf16)
    embn = embedding.astype(bf16)
    emb_packed = lax.bitcast_convert_type(
        embn.reshape(K, D // 2, 2), jnp.int32)
    e2h = (0.5 * jnp.sum(embedding * embedding, axis=1)).reshape(1, K)

    ze, idx = _encode_stage(xf, ew1t, eb1, ew2t, eb2, ew3t, eb3, embt, e2h,
                            N, A, H, D, K)

    ns = SC_ROWS
    zq_packed = _gather_rows(emb_packed, idx[:ns].reshape(ns))
    zq_sc = lax.bitcast_convert_type(zq_packed, bf16).reshape(ns, D)

    dws = (dw1t, db1.reshape(1, H), dw2t, db2.reshape(1, H),
           dw3t, db3.reshape(1, A))
    out_oh, l_oh = _decode_onehot(idx[ns:], ze[ns:], embn, dws,
                                  N - ns, A, H, D, K)
    out_sc, l_sc = _decode_zq(zq_sc, ze[:ns], dws, ns, A, H, D)

    out = jnp.concatenate([out_sc, out_oh], axis=0)
    loss = (jnp.sum(l_sc) + jnp.sum(l_oh)) / (N * D)
    return (out.reshape(Bb, Tt, A), loss, loss,
            idx.reshape(Bb, Tt).astype(jnp.int32))


def _full(shape):
    return pl.BlockSpec(shape, lambda i: tuple(0 for _ in shape))


def _encode_stage(xf, ew1t, eb1, ew2t, eb2, ew3t, eb3, embt, e2h,
                  N, A, H, D, K):
    BM = 256
    grid = N // BM
    return pl.pallas_call(
        _enc_vq_kernel,
        grid=(grid,),
        in_specs=[
            pl.BlockSpec((BM, A), lambda i: (i, 0)),
            _full((A, H)),
            _full((1, H)),
            _full((H, H)),
            _full((1, H)),
            _full((H, D)),
            _full((1, D)),
            _full((D, K)),
            _full((1, K)),
        ],
        out_specs=[
            pl.BlockSpec((BM, D), lambda i: (i, 0)),
            pl.BlockSpec((BM, 1), lambda i: (i, 0)),
        ],
        out_shape=[
            jax.ShapeDtypeStruct((N, D), jnp.float32),
            jax.ShapeDtypeStruct((N, 1), jnp.int32),
        ],
        compiler_params=pltpu.CompilerParams(
            dimension_semantics=("parallel",)),
    )(xf, ew1t, eb1.reshape(1, H), ew2t, eb2.reshape(1, H),
      ew3t, eb3.reshape(1, D), embt, e2h)


def _decode_zq(zq, ze, dws, n, A, H, D):
    BM = 512
    grid = n // BM
    return pl.pallas_call(
        _dec_zq_kernel,
        grid=(grid,),
        in_specs=[
            pl.BlockSpec((BM, D), lambda i: (i, 0)),
            pl.BlockSpec((BM, D), lambda i: (i, 0)),
            _full((D, H)), _full((1, H)), _full((H, H)), _full((1, H)),
            _full((H, A)), _full((1, A)),
        ],
        out_specs=[
            pl.BlockSpec((BM, A), lambda i: (i, 0)),
            pl.BlockSpec((1, 1, D), lambda i: (i, 0, 0)),
        ],
        out_shape=[
            jax.ShapeDtypeStruct((n, A), jnp.float32),
            jax.ShapeDtypeStruct((grid, 1, D), jnp.float32),
        ],
        compiler_params=pltpu.CompilerParams(
            dimension_semantics=("parallel",)),
    )(zq, ze, *dws)


def _decode_onehot(idx, ze, embn, dws, n, A, H, D, K):
    BM = 512
    grid = n // BM
    return pl.pallas_call(
        _dec_onehot_kernel,
        grid=(grid,),
        in_specs=[
            pl.BlockSpec((BM, 1), lambda i: (i, 0)),
            pl.BlockSpec((BM, D), lambda i: (i, 0)),
            _full((K, D)),
            _full((D, H)), _full((1, H)), _full((H, H)), _full((1, H)),
            _full((H, A)), _full((1, A)),
        ],
        out_specs=[
            pl.BlockSpec((BM, A), lambda i: (i, 0)),
            pl.BlockSpec((1, 1, D), lambda i: (i, 0, 0)),
        ],
        out_shape=[
            jax.ShapeDtypeStruct((n, A), jnp.float32),
            jax.ShapeDtypeStruct((grid, 1, D), jnp.float32),
        ],
        compiler_params=pltpu.CompilerParams(
            dimension_semantics=("parallel",)),
    )(idx, ze, embn, *dws)


# speedup vs baseline: 1.7687x; 1.0344x over previous
import functools

import jax
import jax.numpy as jnp
from jax import lax
from jax.experimental import pallas as pl
from jax.experimental.pallas import tpu as pltpu
from jax.experimental.pallas import tpu_sc as plsc

SC_ROWS = 512


def _enc_vq_kernel(x_ref, w1_ref, b1_ref, w2_ref, b2_ref, w3_ref, b3_ref,
                   embt_ref, e2h_ref, ze_ref, idx_ref):
    h = jnp.dot(x_ref[...], w1_ref[...], preferred_element_type=jnp.float32)
    h = jnp.maximum(h + b1_ref[...], 0.0)
    h = jnp.dot(h.astype(jnp.bfloat16), w2_ref[...],
                preferred_element_type=jnp.float32)
    h = jnp.maximum(h + b2_ref[...], 0.0)
    z = jnp.dot(h.astype(jnp.bfloat16), w3_ref[...],
                preferred_element_type=jnp.float32)
    z = z + b3_ref[...]
    ze_ref[...] = z
    mm = jnp.dot(z.astype(jnp.bfloat16), embt_ref[...],
                 preferred_element_type=jnp.float32)
    s = e2h_ref[...] - mm
    idx = jnp.argmin(s, axis=1).astype(jnp.int32)
    idx_ref[...] = idx.reshape(idx_ref.shape)


def _dec_body(zq, ze_ref, w1_ref, b1_ref, w2_ref, b2_ref, w3_ref, b3_ref,
              out_ref, ls_ref):
    d = ze_ref[...] - zq.astype(jnp.float32)
    ls_ref[...] = jnp.sum(d * d, axis=0, keepdims=True).reshape(ls_ref.shape)
    h = jnp.dot(zq, w1_ref[...], preferred_element_type=jnp.float32)
    h = jnp.maximum(h + b1_ref[...], 0.0)
    h = jnp.dot(h.astype(jnp.bfloat16), w2_ref[...],
                preferred_element_type=jnp.float32)
    h = jnp.maximum(h + b2_ref[...], 0.0)
    o = jnp.dot(h.astype(jnp.bfloat16), w3_ref[...],
                preferred_element_type=jnp.float32)
    out_ref[...] = o + b3_ref[...]


def _dec_zq_kernel(zq_ref, ze_ref, w1_ref, b1_ref, w2_ref, b2_ref,
                   w3_ref, b3_ref, out_ref, ls_ref):
    _dec_body(zq_ref[...], ze_ref, w1_ref, b1_ref, w2_ref, b2_ref,
              w3_ref, b3_ref, out_ref, ls_ref)


def _dec_onehot_kernel(idx_ref, ze_ref, embn_ref, w1_ref, b1_ref, w2_ref,
                       b2_ref, w3_ref, b3_ref, out_ref, ls_ref):
    bm, _ = idx_ref.shape
    kk = embn_ref.shape[0]
    iota = lax.broadcasted_iota(jnp.int32, (bm, kk), 1)
    oh = (iota == idx_ref[...]).astype(jnp.bfloat16)
    zq = jnp.dot(oh, embn_ref[...], preferred_element_type=jnp.float32)
    _dec_body(zq.astype(jnp.bfloat16), ze_ref, w1_ref, b1_ref, w2_ref,
              b2_ref, w3_ref, b3_ref, out_ref, ls_ref)


def _gather_rows(table, idx):
    V, ln = table.shape
    (n,) = idx.shape
    info = pltpu.get_tpu_info().sparse_core
    nw = info.num_cores * info.num_subcores
    b_per_w = n // nw
    mesh = plsc.VectorSubcoreMesh(core_axis_name="c", subcore_axis_name="s")

    @functools.partial(
        pl.kernel, mesh=mesh,
        out_type=jax.ShapeDtypeStruct((n, ln), table.dtype),
        scratch_types=[
            pltpu.VMEM((b_per_w,), jnp.int32),
            pltpu.VMEM((b_per_w, ln), table.dtype),
            pltpu.SemaphoreType.DMA,
        ],
    )
    def k(table_hbm, idx_hbm, out_hbm, idx_v, rows_v, sem):
        wid = lax.axis_index("s") * info.num_cores + lax.axis_index("c")
        base = wid * b_per_w
        pltpu.sync_copy(idx_hbm.at[pl.ds(base, b_per_w)], idx_v)
        pltpu.async_copy(table_hbm.at[idx_v], rows_v, sem).wait()
        pltpu.sync_copy(rows_v, out_hbm.at[pl.ds(base, b_per_w)])

    return k(table, idx)


def kernel(x, embedding, ew1, eb1, ew2, eb2, ew3, eb3,
           dw1, db1, dw2, db2, dw3, db3):
    Bb, Tt, A = x.shape
    N = Bb * Tt
    K, D = embedding.shape
    H = ew1.shape[0]
    bf16 = jnp.bfloat16

    xf = x.reshape(N, A).astype(bf16)
    ew1t = ew1.T.astype(bf16)
    ew2t = ew2.T.astype(bf16)
    ew3t = ew3.T.astype(bf16)
    dw1t = dw1.T.astype(bf16)
    dw2t = dw2.T.astype(bf16)
    dw3t = dw3.T.astype(bf16)
    embt = embedding.T.astype(bf16)
    embn = embedding.astype(bf16)
    emb_packed = lax.bitcast_convert_type(
        embn.reshape(K, D // 2, 2), jnp.int32)
    e2h = (0.5 * jnp.sum(embedding * embedding, axis=1)).reshape(1, K)

    ze, idx = _encode_stage(xf, ew1t, eb1, ew2t, eb2, ew3t, eb3, embt, e2h,
                            N, A, H, D, K)

    ns = SC_ROWS
    zq_packed = _gather_rows(emb_packed, idx[:ns].reshape(ns))
    zq_sc = lax.bitcast_convert_type(zq_packed, bf16).reshape(ns, D)

    dws = (dw1t, db1.reshape(1, H), dw2t, db2.reshape(1, H),
           dw3t, db3.reshape(1, A))
    out_oh, l_oh = _decode_onehot(idx[ns:], ze[ns:], embn, dws,
                                  N - ns, A, H, D, K)
    out_sc, l_sc = _decode_zq(zq_sc, ze[:ns], dws, ns, A, H, D)

    out = jnp.concatenate([out_sc, out_oh], axis=0)
    loss = (jnp.sum(l_sc) + jnp.sum(l_oh)) / (N * D)
    return (out.reshape(Bb, Tt, A), loss, loss,
            idx.reshape(Bb, Tt).astype(jnp.int32))


def _full(shape):
    return pl.BlockSpec(shape, lambda i: tuple(0 for _ in shape))


def _encode_stage(xf, ew1t, eb1, ew2t, eb2, ew3t, eb3, embt, e2h,
                  N, A, H, D, K):
    BM = 256
    grid = N // BM
    return pl.pallas_call(
        _enc_vq_kernel,
        grid=(grid,),
        in_specs=[
            pl.BlockSpec((BM, A), lambda i: (i, 0)),
            _full((A, H)),
            _full((1, H)),
            _full((H, H)),
            _full((1, H)),
            _full((H, D)),
            _full((1, D)),
            _full((D, K)),
            _full((1, K)),
        ],
        out_specs=[
            pl.BlockSpec((BM, D), lambda i: (i, 0)),
            pl.BlockSpec((BM, 1), lambda i: (i, 0)),
        ],
        out_shape=[
            jax.ShapeDtypeStruct((N, D), jnp.float32),
            jax.ShapeDtypeStruct((N, 1), jnp.int32),
        ],
        compiler_params=pltpu.CompilerParams(
            dimension_semantics=("parallel",)),
    )(xf, ew1t, eb1.reshape(1, H), ew2t, eb2.reshape(1, H),
      ew3t, eb3.reshape(1, D), embt, e2h)


def _decode_zq(zq, ze, dws, n, A, H, D):
    BM = 512
    grid = n // BM
    return pl.pallas_call(
        _dec_zq_kernel,
        grid=(grid,),
        in_specs=[
            pl.BlockSpec((BM, D), lambda i: (i, 0)),
            pl.BlockSpec((BM, D), lambda i: (i, 0)),
            _full((D, H)), _full((1, H)), _full((H, H)), _full((1, H)),
            _full((H, A)), _full((1, A)),
        ],
        out_specs=[
            pl.BlockSpec((BM, A), lambda i: (i, 0)),
            pl.BlockSpec((1, 1, D), lambda i: (i, 0, 0)),
        ],
        out_shape=[
            jax.ShapeDtypeStruct((n, A), jnp.float32),
            jax.ShapeDtypeStruct((grid, 1, D), jnp.float32),
        ],
        compiler_params=pltpu.CompilerParams(
            dimension_semantics=("parallel",)),
    )(zq, ze, *dws)


def _decode_onehot(idx, ze, embn, dws, n, A, H, D, K):
    BM = 512
    grid = n // BM
    return pl.pallas_call(
        _dec_onehot_kernel,
        grid=(grid,),
        in_specs=[
            pl.BlockSpec((BM, 1), lambda i: (i, 0)),
            pl.BlockSpec((BM, D), lambda i: (i, 0)),
            _full((K, D)),
            _full((D, H)), _full((1, H)), _full((H, H)), _full((1, H)),
            _full((H, A)), _full((1, A)),
        ],
        out_specs=[
            pl.BlockSpec((BM, A), lambda i: (i, 0)),
            pl.BlockSpec((1, 1, D), lambda i: (i, 0, 0)),
        ],
        out_shape=[
            jax.ShapeDtypeStruct((n, A), jnp.float32),
            jax.ShapeDtypeStruct((grid, 1, D), jnp.float32),
        ],
        compiler_params=pltpu.CompilerParams(
            dimension_semantics=("parallel",)),
    )(idx, ze, embn, *dws)
